# speedup vs baseline: 2.1326x; 2.1326x over previous
import jax
import jax.numpy as jnp
from jax.experimental import pallas as pl
from jax.experimental.pallas import tpu as pltpu


def _reduce_kernel(x_ref, wr_ref, br_ref, acc_ref):
    i = pl.program_id(0)
    xb = x_ref[...]
    logits = jax.lax.dot_general(
        xb, wr_ref[...], (((1,), (1,)), ((), ())),
        preferred_element_type=jnp.float32)
    logits = logits + br_ref[...]
    m = jnp.max(logits, axis=1, keepdims=True)
    denom = jnp.sum(jnp.exp(logits - m), axis=1, keepdims=True)
    G = 1.0 / denom
    u = jax.lax.dot_general(
        G, xb, (((0,), (0,)), ((), ())),
        preferred_element_type=jnp.float32)
    gsum = jnp.sum(G, axis=0, keepdims=True)
    part = jnp.concatenate(
        [u, jnp.broadcast_to(gsum, u.shape)], axis=0)

    @pl.when(i == 0)
    def _():
        acc_ref[...] = part

    @pl.when(i != 0)
    def _():
        acc_ref[...] += part


def _bcast_kernel(acc_ref, we_ref, be_ref, out_ref):
    u = acc_ref[0:1, :]
    g = acc_ref[1, 0]
    vc = jax.lax.dot_general(
        u, we_ref[...], (((1,), (1,)), ((), ())),
        preferred_element_type=jnp.float32)
    vc = vc + g * be_ref[...]
    out_ref[...] = jnp.broadcast_to(vc, out_ref.shape)


def kernel(x, W_r, b_r, W_e, b_e):
    b, s, h = x.shape
    bs = b * s
    e = W_r.shape[0]
    xf = x.reshape(bs, h)
    br2 = b_r.reshape(1, e)
    be2 = b_e.reshape(1, h)

    TB = 512
    acc = pl.pallas_call(
        _reduce_kernel,
        grid=(bs // TB,),
        in_specs=[
            pl.BlockSpec((TB, h), lambda i: (i, 0)),
            pl.BlockSpec((e, h), lambda i: (0, 0)),
            pl.BlockSpec((1, e), lambda i: (0, 0)),
        ],
        out_specs=pl.BlockSpec((2, h), lambda i: (0, 0)),
        out_shape=jax.ShapeDtypeStruct((2, h), jnp.float32),
        compiler_params=pltpu.CompilerParams(
            dimension_semantics=("arbitrary",)),
    )(xf, W_r, br2)

    CB = 256
    yflat = pl.pallas_call(
        _bcast_kernel,
        grid=(h // CB,),
        in_specs=[
            pl.BlockSpec((2, h), lambda c: (0, 0)),
            pl.BlockSpec((CB, h), lambda c: (c, 0)),
            pl.BlockSpec((1, CB), lambda c: (0, c)),
        ],
        out_specs=pl.BlockSpec((bs, CB), lambda c: (0, c)),
        out_shape=jax.ShapeDtypeStruct((bs, h), jnp.float32),
        compiler_params=pltpu.CompilerParams(
            dimension_semantics=("arbitrary",)),
    )(acc, W_e, be2)

    return yflat.reshape(b, s, h)
